# Initial kernel scaffold; baseline (speedup 1.0000x reference)
import jax
import jax.numpy as jnp
from jax.experimental import pallas as pl


def kernel(flow_list, event_list, pol_mask):
    raise NotImplementedError("write your pallas kernel here")


# speedup vs baseline: 44.5650x; 44.5650x over previous
import functools

import jax
import jax.numpy as jnp
from jax import lax
from jax.experimental import pallas as pl
from jax.experimental.pallas import tpu as pltpu
from jax.experimental.pallas import tpu_sc as plsc

B = 4
N = 100000
H = 480
W = 640
HW = H * W
GRID2 = 2 * HW
FLOW_SCALING = float(max(H, W))

NC = 2
NS = 16
NE = 6272
NP = NS * NE
NG = NE // 128
TSL = GRID2 // NS
ZCH = 7680

_f32 = jnp.float32
_i32 = jnp.int32


def _sc_body(ts_h, ys_h, xs_h, sel_h, fy_h, fx_h, out_h,
             ts_v, ys_v, xs_v, sel_v, gi_v, fy_v, fx_v,
             ist, wst, tst, zz, accw, acct, sem):
    c = lax.axis_index("c")
    s = lax.axis_index("s")
    off = s * NE
    zoff = s * TSL

    def _zfill(i, carry):
        zz[pl.ds(i * 16, 16)] = jnp.zeros((16,), _f32)
        return carry
    lax.fori_loop(0, ZCH // 16, _zfill, 0)

    for bl in range(2):
        b = c * 2 + bl
        pltpu.sync_copy(ts_h.at[b, pl.ds(off, NE)], ts_v)
        pltpu.sync_copy(ys_h.at[b, pl.ds(off, NE)], ys_v)
        pltpu.sync_copy(xs_h.at[b, pl.ds(off, NE)], xs_v)
        pltpu.sync_copy(sel_h.at[b, pl.ds(off, NE)], sel_v)

        base = b * HW

        def _gidx(i, carry):
            yv = ys_v[pl.ds(i * 16, 16)].astype(_i32)
            xv = xs_v[pl.ds(i * 16, 16)].astype(_i32)
            g = base + yv * W + xv
            g = jnp.minimum(jnp.maximum(g, 0), B * HW - 1)
            gi_v[pl.ds(i * 16, 16)] = g
            return carry
        lax.fori_loop(0, NE // 16, _gidx, 0)

        pltpu.async_copy(fy_h.at[gi_v], fy_v, sem).wait()
        pltpu.async_copy(fx_h.at[gi_v], fx_v, sem).wait()

        for d in range(2):
            tref = 1.0 if d == 0 else 0.0
            for k in range(TSL // ZCH):
                pltpu.sync_copy(zz, accw.at[pl.ds(zoff + k * ZCH, ZCH)])
                pltpu.sync_copy(zz, acct.at[pl.ds(zoff + k * ZCH, ZCH)])
            plsc.subcore_barrier()

            def _group(g, carry):
                for sub in range(8):
                    e = g * 128 + sub * 16
                    tsv = ts_v[pl.ds(e, 16)]
                    yv = ys_v[pl.ds(e, 16)]
                    xv = xs_v[pl.ds(e, 16)]
                    sv = sel_v[pl.ds(e, 16)]
                    fyv = fy_v[pl.ds(e, 16)]
                    fxv = fx_v[pl.ds(e, 16)]
                    dt = tref - tsv
                    wy = yv + dt * fyv * FLOW_SCALING
                    wx = xv + dt * fxv * FLOW_SCALING
                    ty = wy.astype(_i32).astype(_f32)
                    tx = wx.astype(_i32).astype(_f32)
                    fy0 = jnp.where(ty > wy, ty - 1.0, ty)
                    fx0 = jnp.where(tx > wx, tx - 1.0, tx)
                    ry = wy - fy0
                    rx = wx - fx0
                    iy0 = fy0.astype(_i32)
                    ix0 = fx0.astype(_i32)
                    iy1 = iy0 + 1
                    ix1 = ix0 + 1
                    y0ok = (iy0 >= 0) & (iy0 < H)
                    y1ok = (iy1 >= 0) & (iy1 < H)
                    x0ok = (ix0 >= 0) & (ix0 < W)
                    x1ok = (ix1 >= 0) & (ix1 < W)
                    wy0 = 1.0 - ry
                    wx0 = 1.0 - rx
                    tval = tsv if d == 0 else 1.0 - tsv
                    seli = sv.astype(_i32) * HW
                    corners = (
                        (y0ok, iy0, wy0, x0ok, ix0, wx0),
                        (y0ok, iy0, wy0, x1ok, ix1, rx),
                        (y1ok, iy1, ry, x0ok, ix0, wx0),
                        (y1ok, iy1, ry, x1ok, ix1, rx),
                    )
                    for ci, (yok, iy, wyv, xok, ix, wxv) in enumerate(corners):
                        ok = yok & xok
                        w = jnp.where(ok, wyv * wxv, 0.0)
                        row = jnp.where(ok, seli + iy * W + ix, 0)
                        ist[ci, pl.ds(sub * 16, 16)] = row
                        wst[ci, pl.ds(sub * 16, 16)] = w
                        tst[ci, pl.ds(sub * 16, 16)] = w * tval
                for ci in range(4):
                    pltpu.sync_copy(wst.at[ci], accw.at[ist.at[ci]], add=True)
                    pltpu.sync_copy(tst.at[ci], acct.at[ist.at[ci]], add=True)
                return carry
            lax.fori_loop(0, NG, _group, 0)
            plsc.subcore_barrier()

            pltpu.sync_copy(accw.at[pl.ds(zoff, TSL)],
                            out_h.at[b, d, 0, pl.ds(zoff, TSL)])
            pltpu.sync_copy(acct.at[pl.ds(zoff, TSL)],
                            out_h.at[b, d, 1, pl.ds(zoff, TSL)])
            plsc.subcore_barrier()


_sc_warp = functools.partial(
    pl.kernel,
    out_type=jax.ShapeDtypeStruct((B, 2, 2, GRID2), _f32),
    mesh=plsc.VectorSubcoreMesh(core_axis_name="c", subcore_axis_name="s"),
    scratch_types=[
        pltpu.VMEM((NE,), _f32),
        pltpu.VMEM((NE,), _f32),
        pltpu.VMEM((NE,), _f32),
        pltpu.VMEM((NE,), _f32),
        pltpu.VMEM((NE,), _i32),
        pltpu.VMEM((NE,), _f32),
        pltpu.VMEM((NE,), _f32),
        pltpu.VMEM((4, 128), _i32),
        pltpu.VMEM((4, 128), _f32),
        pltpu.VMEM((4, 128), _f32),
        pltpu.VMEM((ZCH,), _f32),
        pltpu.VMEM_SHARED((GRID2,), _f32),
        pltpu.VMEM_SHARED((GRID2,), _f32),
        pltpu.SemaphoreType.DMA,
    ],
)(_sc_body)


def _reduce_body(w_ref, t_ref, out_ref):
    i = pl.program_id(0)
    j = pl.program_id(1)

    @pl.when((i == 0) & (j == 0))
    def _():
        out_ref[...] = jnp.zeros((1, 1), _f32)

    w = w_ref[0, 0]
    t = t_ref[0, 0]
    r = t / (w + 1e-9)
    out_ref[...] += jnp.sum(r * r).reshape(1, 1)


def _smooth_body(f_ref, out_ref):
    i = pl.program_id(0)

    @pl.when(i == 0)
    def _():
        out_ref[...] = jnp.zeros((1, 1), _f32)

    f = f_ref[0, 0]
    dx = f[:-1, :] - f[1:, :]
    dy = f[:, :-1] - f[:, 1:]
    out_ref[...] += (jnp.sum(jnp.sqrt(dx * dx + 1e-6))
                     + jnp.sum(jnp.sqrt(dy * dy + 1e-6))).reshape(1, 1)


RRED = 1200
NJ = (GRID2 // 128) // RRED


def kernel(flow_list, event_list, pol_mask):
    ts = event_list[:, :, 0]
    ys = event_list[:, :, 1]
    xs = event_list[:, :, 2]
    sel = pol_mask[:, :, 1]
    pad = NP - N
    ts_p = jnp.pad(ts, ((0, 0), (0, pad)))
    ys_p = jnp.pad(ys, ((0, 0), (0, pad)), constant_values=-1e6)
    xs_p = jnp.pad(xs, ((0, 0), (0, pad)), constant_values=-1e6)
    sel_p = jnp.pad(sel, ((0, 0), (0, pad)))
    flowy = flow_list[:, 1].reshape(B * HW)
    flowx = flow_list[:, 0].reshape(B * HW)

    acc = _sc_warp(ts_p, ys_p, xs_p, sel_p, flowy, flowx)
    accr = acc.reshape(2 * B, 2, GRID2 // 128, 128)

    iwe_loss = pl.pallas_call(
        _reduce_body,
        grid=(2 * B, NJ),
        in_specs=[
            pl.BlockSpec((1, 1, RRED, 128), lambda i, j: (i, 0, j, 0)),
            pl.BlockSpec((1, 1, RRED, 128), lambda i, j: (i, 1, j, 0)),
        ],
        out_specs=pl.BlockSpec((1, 1), lambda i, j: (0, 0)),
        out_shape=jax.ShapeDtypeStruct((1, 1), _f32),
    )(accr, accr)

    smooth_loss = pl.pallas_call(
        _smooth_body,
        grid=(2 * B,),
        in_specs=[pl.BlockSpec((1, 1, H, W), lambda i: (i // 2, i % 2, 0, 0))],
        out_specs=pl.BlockSpec((1, 1), lambda i: (0, 0)),
        out_shape=jax.ShapeDtypeStruct((1, 1), _f32),
    )(flow_list)

    return (iwe_loss[0, 0] + smooth_loss[0, 0]).astype(_f32)
